# Initial kernel scaffold; baseline (speedup 1.0000x reference)
import jax
import jax.numpy as jnp
from jax.experimental import pallas as pl


def kernel(h, W_w, b_w, W_n, b_n, noise):
    raise NotImplementedError("write your pallas kernel here")


# speedup vs baseline: 2.4261x; 2.4261x over previous
import functools

import jax
import jax.numpy as jnp
from jax.experimental import pallas as pl
from jax.experimental.pallas import tpu as pltpu

N = 16384
D = 2048
NEXP = 16
BLK = 512


def _router_block(h_ref, ww_ref, wn_ref, bw_ref, bn_ref, noise_ref,
                  sparse_ref, ix_ref, full_ref):
    h = h_ref[...]
    logits = jax.lax.dot_general(
        h, ww_ref[...], (((1,), (1,)), ((), ())),
        preferred_element_type=jnp.float32) + bw_ref[...]
    nlin = jax.lax.dot_general(
        h, wn_ref[...], (((1,), (1,)), ((), ())),
        preferred_element_type=jnp.float32) + bn_ref[...]
    noisy = logits + noise_ref[...] * jax.nn.softplus(nlin)

    m = jnp.max(noisy, axis=1, keepdims=True)
    e = jnp.exp(noisy - m)
    full_ref[...] = e / jnp.sum(e, axis=1, keepdims=True)

    iota = jax.lax.broadcasted_iota(jnp.int32, noisy.shape, 1)
    ix1 = jnp.min(jnp.where(noisy == m, iota, NEXP), axis=1, keepdims=True)
    masked = jnp.where(iota == ix1, -jnp.inf, noisy)
    m2 = jnp.max(masked, axis=1, keepdims=True)
    ix2 = jnp.min(jnp.where(masked == m2, iota, NEXP), axis=1, keepdims=True)

    e2 = jnp.exp(m2 - m)
    p1 = 1.0 / (1.0 + e2)
    p2 = e2 * p1
    sparse_ref[...] = jnp.where(iota == ix1, p1,
                                jnp.where(iota == ix2, p2, 0.0))
    ix_ref[...] = jnp.concatenate([ix1, ix2], axis=1)


@jax.jit
def _router(h, W_w, b_w, W_n, b_n, noise):
    grid = (N // BLK,)
    return pl.pallas_call(
        _router_block,
        grid=grid,
        in_specs=[
            pl.BlockSpec((BLK, D), lambda i: (i, 0)),
            pl.BlockSpec((NEXP, D), lambda i: (0, 0)),
            pl.BlockSpec((NEXP, D), lambda i: (0, 0)),
            pl.BlockSpec((1, NEXP), lambda i: (0, 0)),
            pl.BlockSpec((1, NEXP), lambda i: (0, 0)),
            pl.BlockSpec((BLK, NEXP), lambda i: (i, 0)),
        ],
        out_specs=[
            pl.BlockSpec((BLK, NEXP), lambda i: (i, 0)),
            pl.BlockSpec((BLK, 2), lambda i: (i, 0)),
            pl.BlockSpec((BLK, NEXP), lambda i: (i, 0)),
        ],
        out_shape=[
            jax.ShapeDtypeStruct((N, NEXP), jnp.float32),
            jax.ShapeDtypeStruct((N, 2), jnp.int32),
            jax.ShapeDtypeStruct((N, NEXP), jnp.float32),
        ],
    )(h, W_w, W_n, b_w.reshape(1, NEXP), b_n.reshape(1, NEXP), noise)


def kernel(h, W_w, b_w, W_n, b_n, noise):
    sparse, ix, full = _router(h, W_w, b_w, W_n, b_n, noise)
    return (sparse, ix, full)
